# baseline (speedup 1.0000x reference)
import functools

import jax
import jax.numpy as jnp
from jax.experimental import pallas as pl
from jax.experimental.pallas import tpu as pltpu

NUM_EXPERTS = 64
TOP_K = 8
T_BLK = 64


def _gating_kernel(x_ref, w_ref, b_ref, sw_ref, idx_ref, acc_ref, *, n_blocks, seq_len):
    t = pl.program_id(0)

    @pl.when(t == 0)
    def _init():
        acc_ref[...] = jnp.zeros_like(acc_ref)

    acc_ref[...] += jnp.sum(x_ref[...], axis=1)

    @pl.when(t == n_blocks - 1)
    def _finish():
        pooled = acc_ref[...] * (1.0 / seq_len)
        logits = jax.lax.dot_general(
            pooled, w_ref[...], (((1,), (1,)), ((), ())),
            precision=jax.lax.Precision.HIGHEST,
            preferred_element_type=jnp.float32,
        ) + b_ref[...]

        e_iota = jax.lax.broadcasted_iota(jnp.int32, logits.shape, 1)
        vals = logits
        top_vals = []
        top_idx = []
        for _ in range(TOP_K):
            m = jnp.max(vals, axis=1, keepdims=True)
            i = jnp.min(jnp.where(vals == m, e_iota, NUM_EXPERTS),
                        axis=1, keepdims=True)
            top_vals.append(m)
            top_idx.append(i)
            vals = jnp.where(e_iota == i, -jnp.inf, vals)

        tv = jnp.concatenate(top_vals, axis=1)
        ex = jnp.exp(tv - tv[:, :1])
        probs = ex / jnp.sum(ex, axis=1, keepdims=True)

        sparse = jnp.zeros_like(logits)
        for k in range(TOP_K):
            sparse += jnp.where(e_iota == top_idx[k], probs[:, k:k + 1], 0.0)

        sw_ref[...] = sparse
        idx_ref[...] = jnp.concatenate(top_idx, axis=1)


@jax.jit
def kernel(x, W, b):
    B, T, D = x.shape
    n_blocks = T // T_BLK
    grid = (n_blocks,)
    sw, idx = pl.pallas_call(
        functools.partial(_gating_kernel, n_blocks=n_blocks, seq_len=T),
        grid=grid,
        in_specs=[
            pl.BlockSpec((B, T_BLK, D), lambda t: (0, t, 0)),
            pl.BlockSpec((NUM_EXPERTS, D), lambda t: (0, 0)),
            pl.BlockSpec((1, NUM_EXPERTS), lambda t: (0, 0)),
        ],
        out_specs=[
            pl.BlockSpec((B, NUM_EXPERTS), lambda t: (0, 0)),
            pl.BlockSpec((B, TOP_K), lambda t: (0, 0)),
        ],
        out_shape=[
            jax.ShapeDtypeStruct((B, NUM_EXPERTS), jnp.float32),
            jax.ShapeDtypeStruct((B, TOP_K), jnp.int32),
        ],
        scratch_shapes=[pltpu.VMEM((B, D), jnp.float32)],
    )(x, W, b.reshape(1, NUM_EXPERTS))
    return (sw, idx)


# speedup vs baseline: 1.3984x; 1.3984x over previous
import functools

import jax
import jax.numpy as jnp
from jax.experimental import pallas as pl
from jax.experimental.pallas import tpu as pltpu

NUM_EXPERTS = 64
TOP_K = 8
T_BLK = 128


def _gating_kernel(x_ref, w_ref, b_ref, sw_ref, idx_ref, acc_ref, *, n_blocks, seq_len):
    t = pl.program_id(0)

    @pl.when(t == 0)
    def _init():
        acc_ref[...] = jnp.zeros_like(acc_ref)

    acc_ref[...] += jnp.sum(x_ref[...], axis=1)

    @pl.when(t == n_blocks - 1)
    def _finish():
        pooled = acc_ref[...] * (1.0 / seq_len)
        logits = jax.lax.dot_general(
            pooled, w_ref[...], (((1,), (1,)), ((), ())),
            precision=jax.lax.Precision.HIGHEST,
            preferred_element_type=jnp.float32,
        ) + b_ref[...]

        e_iota = jax.lax.broadcasted_iota(jnp.int32, logits.shape, 1)
        vals = logits
        top_vals = []
        top_idx = []
        for _ in range(TOP_K):
            m = jnp.max(vals, axis=1, keepdims=True)
            i = jnp.min(jnp.where(vals == m, e_iota, NUM_EXPERTS),
                        axis=1, keepdims=True)
            top_vals.append(m)
            top_idx.append(i)
            vals = jnp.where(e_iota == i, -jnp.inf, vals)

        tv = jnp.concatenate(top_vals, axis=1)
        ex = jnp.exp(tv - tv[:, :1])
        probs = ex / jnp.sum(ex, axis=1, keepdims=True)

        sparse = jnp.zeros_like(logits)
        for k in range(TOP_K):
            sparse += jnp.where(e_iota == top_idx[k], probs[:, k:k + 1], 0.0)

        sw_ref[...] = sparse
        idx_ref[...] = jnp.concatenate(top_idx, axis=1)


@jax.jit
def kernel(x, W, b):
    B, T, D = x.shape
    n_blocks = T // T_BLK
    grid = (n_blocks,)
    sw, idx = pl.pallas_call(
        functools.partial(_gating_kernel, n_blocks=n_blocks, seq_len=T),
        grid=grid,
        in_specs=[
            pl.BlockSpec((B, T_BLK, D), lambda t: (0, t, 0)),
            pl.BlockSpec((NUM_EXPERTS, D), lambda t: (0, 0)),
            pl.BlockSpec((1, NUM_EXPERTS), lambda t: (0, 0)),
        ],
        out_specs=[
            pl.BlockSpec((B, NUM_EXPERTS), lambda t: (0, 0)),
            pl.BlockSpec((B, TOP_K), lambda t: (0, 0)),
        ],
        out_shape=[
            jax.ShapeDtypeStruct((B, NUM_EXPERTS), jnp.float32),
            jax.ShapeDtypeStruct((B, TOP_K), jnp.int32),
        ],
        scratch_shapes=[pltpu.VMEM((B, D), jnp.float32)],
    )(x, W, b.reshape(1, NUM_EXPERTS))
    return (sw, idx)
